# baseline (device time: 26527 ns/iter reference)
import jax
import jax.numpy as jnp
from jax import lax
from jax.experimental import pallas as pl
from jax.experimental.pallas import tpu as pltpu

M, N = 1024, 1024
MESH = pl.DeviceIdType.MESH


def kernel(x):
    x_bf = x.reshape(M, N).astype(jnp.bfloat16)

    def body(x_ref, out_ref, r1a, r1b, r2a, r2b, r4a, r4b,
             send_sems, recv_sems):
        mx = lax.axis_index("x")
        my = lax.axis_index("y")
        xn = (1 - mx, my)
        yn = (mx, 1 - my)

        def rdma(src, dst, i, dev):
            return pltpu.make_async_remote_copy(
                src_ref=src, dst_ref=dst,
                send_sem=send_sems.at[i], recv_sem=recv_sems.at[i],
                device_id=dev, device_id_type=MESH,
            )

        class H:
            pass

        A = H()
        A.outer_sel = mx
        A.inner_sel = my
        A.base = 0
        A.outer_nbr = xn
        A.inner_nbr = yn
        A.r1, A.r2, A.r4 = r1a, r2a, r4a
        A.sem = 0

        B = H()
        B.outer_sel = my
        B.inner_sel = mx
        B.base = 512
        B.outer_nbr = yn
        B.inner_nbr = xn
        B.r1, B.r2, B.r4 = r1b, r2b, r4b
        B.sem = 12

        for h in (A, B):
            h.base_send = h.base + 256 * (1 - h.outer_sel)
            h.base_keep = h.base + 256 * h.outer_sel
            h.q_send = h.base_keep + 128 * (1 - h.inner_sel)
            h.q_keep = h.base_keep + 128 * h.inner_sel
            h.offs1 = [128 * (1 - h.inner_sel), 128 * (1 - h.inner_sel) + 64,
                       128 * h.inner_sel, 128 * h.inner_sel + 64]

        barrier_sem = pltpu.get_barrier_semaphore()
        for nbr in (xn, yn):
            pl.semaphore_signal(
                barrier_sem, inc=1, device_id=nbr, device_id_type=MESH)
        pl.semaphore_wait(barrier_sem, 2)

        for h in (A, B):
            h.p1 = [rdma(x_ref.at[pl.ds(h.base_send + o, 64)],
                         h.r1.at[pl.ds(o, 64)], h.sem + c, h.outer_nbr)
                    for c, o in enumerate(h.offs1)]
        for c in range(4):
            A.p1[c].start()
            B.p1[c].start()

        for h in (A, B):
            h.p2 = [rdma(out_ref.at[pl.ds(h.q_send + 64 * c, 64)],
                         h.r2.at[pl.ds(64 * c, 64)], h.sem + 4 + c,
                         h.inner_nbr)
                    for c in range(2)]
        for cs in (range(2), range(2, 4)):
            for c in cs:
                for h in (A, B):
                    h.p1[c].wait_recv()
                    o = h.offs1[c]
                    out_ref[pl.ds(h.base_keep + o, 64), :] = (
                        x_ref[pl.ds(h.base_keep + o, 64), :]
                        + h.r1[pl.ds(o, 64), :])
                    if c < 2:
                        h.p2[c].start()

        for h in (A, B):
            h.p3 = [rdma(out_ref.at[pl.ds(h.q_keep + 64 * c, 64)],
                         out_ref.at[pl.ds(h.q_keep + 64 * c, 64)],
                         h.sem + 6 + c, h.inner_nbr)
                    for c in range(2)]
            h.p4k = [rdma(out_ref.at[pl.ds(h.q_keep + 64 * c, 64)],
                          h.r4.at[pl.ds(128 * h.inner_sel + 64 * c, 64)],
                          h.sem + 8 + c, h.outer_nbr)
                     for c in range(2)]
            h.p4s = [rdma(out_ref.at[pl.ds(h.q_send + 64 * c, 64)],
                          h.r4.at[pl.ds(128 * (1 - h.inner_sel) + 64 * c, 64)],
                          h.sem + 10 + c, h.outer_nbr)
                     for c in range(2)]
        for c in range(2):
            for h in (A, B):
                h.p2[c].wait_recv()
                off = h.q_keep + 64 * c
                out_ref[pl.ds(off, 64), :] = (
                    out_ref[pl.ds(off, 64), :] + h.r2[pl.ds(64 * c, 64), :])
                h.p3[c].start()
                h.p4k[c].start()

        for c in range(2):
            for h in (A, B):
                h.p3[c].wait_recv()
                h.p4s[c].start()

        for h in (A, B):
            for c in range(2):
                h.p4k[c].wait_recv()
                o = 128 * h.inner_sel + 64 * c
                out_ref[pl.ds(h.base_send + o, 64), :] = h.r4[pl.ds(o, 64), :]
            for c in range(2):
                h.p4s[c].wait_recv()
                o = 128 * (1 - h.inner_sel) + 64 * c
                out_ref[pl.ds(h.base_send + o, 64), :] = h.r4[pl.ds(o, 64), :]

        for h in (A, B):
            for d in h.p1 + h.p2 + h.p3 + h.p4k + h.p4s:
                d.wait_send()

    return pl.pallas_call(
        body,
        out_shape=jax.ShapeDtypeStruct((M, N), jnp.bfloat16),
        in_specs=[pl.BlockSpec(memory_space=pltpu.VMEM)],
        out_specs=pl.BlockSpec(memory_space=pltpu.VMEM),
        scratch_shapes=[
            pltpu.VMEM((256, N), jnp.bfloat16),
            pltpu.VMEM((256, N), jnp.bfloat16),
            pltpu.VMEM((128, N), jnp.bfloat16),
            pltpu.VMEM((128, N), jnp.bfloat16),
            pltpu.VMEM((256, N), jnp.bfloat16),
            pltpu.VMEM((256, N), jnp.bfloat16),
            pltpu.SemaphoreType.DMA((24,)),
            pltpu.SemaphoreType.DMA((24,)),
        ],
        compiler_params=pltpu.CompilerParams(collective_id=0),
    )(x_bf)


# device time: 25534 ns/iter; 1.0389x vs baseline; 1.0389x over previous
import jax
import jax.numpy as jnp
from jax import lax
from jax.experimental import pallas as pl
from jax.experimental.pallas import tpu as pltpu

M, N = 1024, 1024
MESH = pl.DeviceIdType.MESH


def kernel(x):
    def body(x_ref, out_ref, r1a, r1b, r2a, r2b, r4a, r4b,
             send_sems, recv_sems):
        mx = lax.axis_index("x")
        my = lax.axis_index("y")
        xn = (1 - mx, my)
        yn = (mx, 1 - my)

        def rdma(src, dst, i, dev):
            return pltpu.make_async_remote_copy(
                src_ref=src, dst_ref=dst,
                send_sem=send_sems.at[i], recv_sem=recv_sems.at[i],
                device_id=dev, device_id_type=MESH,
            )

        def cast_rows(off, h):
            out_ref[pl.ds(off, h), :] = x_ref[0, 0, pl.ds(off, h), :].astype(
                jnp.bfloat16)

        def add_rows(off, src_ref, src_off):
            out_ref[pl.ds(off, 64), :] = (
                out_ref[pl.ds(off, 64), :] + src_ref[pl.ds(src_off, 64), :])

        def copy_rows(off, src_ref, src_off):
            out_ref[pl.ds(off, 64), :] = src_ref[pl.ds(src_off, 64), :]

        class H:
            pass

        A = H()
        A.outer_sel = mx
        A.inner_sel = my
        A.base = 0
        A.outer_nbr = xn
        A.inner_nbr = yn
        A.r1, A.r2, A.r4 = r1a, r2a, r4a
        A.sem = 0

        B = H()
        B.outer_sel = my
        B.inner_sel = mx
        B.base = 512
        B.outer_nbr = yn
        B.inner_nbr = xn
        B.r1, B.r2, B.r4 = r1b, r2b, r4b
        B.sem = 12

        for h in (A, B):
            h.base_send = h.base + 256 * (1 - h.outer_sel)
            h.base_keep = h.base + 256 * h.outer_sel
            h.q_send = h.base_keep + 128 * (1 - h.inner_sel)
            h.q_keep = h.base_keep + 128 * h.inner_sel
            h.offs1 = [128 * (1 - h.inner_sel), 128 * (1 - h.inner_sel) + 64,
                       128 * h.inner_sel, 128 * h.inner_sel + 64]

        cast_rows(A.base_send, 256)
        cast_rows(B.base_send, 256)

        barrier_sem = pltpu.get_barrier_semaphore()
        for nbr in (xn, yn):
            pl.semaphore_signal(
                barrier_sem, inc=1, device_id=nbr, device_id_type=MESH)
        pl.semaphore_wait(barrier_sem, 2)

        for h in (A, B):
            h.p1 = [rdma(out_ref.at[pl.ds(h.base_send + o, 64)],
                         h.r1.at[pl.ds(o, 64)], h.sem + c, h.outer_nbr)
                    for c, o in enumerate(h.offs1)]
        for c in range(4):
            A.p1[c].start()
            B.p1[c].start()

        cast_rows(A.base_keep, 256)
        cast_rows(B.base_keep, 256)

        for h in (A, B):
            h.p2 = [rdma(out_ref.at[pl.ds(h.q_send + 64 * c, 64)],
                         h.r2.at[pl.ds(64 * c, 64)], h.sem + 4 + c,
                         h.inner_nbr)
                    for c in range(2)]
        for c in range(2):
            for h in (A, B):
                h.p1[c].wait_recv()
                add_rows(h.base_keep + h.offs1[c], h.r1, h.offs1[c])
                h.p2[c].start()
        for c in range(2, 4):
            for h in (A, B):
                h.p1[c].wait_recv()
                add_rows(h.base_keep + h.offs1[c], h.r1, h.offs1[c])

        for h in (A, B):
            h.p3 = [rdma(out_ref.at[pl.ds(h.q_keep + 64 * c, 64)],
                         out_ref.at[pl.ds(h.q_keep + 64 * c, 64)],
                         h.sem + 6 + c, h.inner_nbr)
                    for c in range(2)]
            h.p4k = [rdma(out_ref.at[pl.ds(h.q_keep + 64 * c, 64)],
                          h.r4.at[pl.ds(128 * h.inner_sel + 64 * c, 64)],
                          h.sem + 8 + c, h.outer_nbr)
                     for c in range(2)]
            h.p4s = [rdma(out_ref.at[pl.ds(h.q_send + 64 * c, 64)],
                          h.r4.at[pl.ds(128 * (1 - h.inner_sel) + 64 * c, 64)],
                          h.sem + 10 + c, h.outer_nbr)
                     for c in range(2)]
        for c in range(2):
            for h in (A, B):
                h.p2[c].wait_recv()
                add_rows(h.q_keep + 64 * c, h.r2, 64 * c)
                h.p3[c].start()
                h.p4k[c].start()

        for c in range(2):
            for h in (A, B):
                h.p3[c].wait_recv()
                h.p4s[c].start()

        for h in (A, B):
            for c in range(2):
                h.p4k[c].wait_recv()
                o = 128 * h.inner_sel + 64 * c
                copy_rows(h.base_send + o, h.r4, o)
            for c in range(2):
                h.p4s[c].wait_recv()
                o = 128 * (1 - h.inner_sel) + 64 * c
                copy_rows(h.base_send + o, h.r4, o)

        for h in (A, B):
            for d in h.p1 + h.p2 + h.p3 + h.p4k + h.p4s:
                d.wait_send()

    return pl.pallas_call(
        body,
        out_shape=jax.ShapeDtypeStruct((M, N), jnp.bfloat16),
        in_specs=[pl.BlockSpec(memory_space=pltpu.VMEM)],
        out_specs=pl.BlockSpec(memory_space=pltpu.VMEM),
        scratch_shapes=[
            pltpu.VMEM((256, N), jnp.bfloat16),
            pltpu.VMEM((256, N), jnp.bfloat16),
            pltpu.VMEM((128, N), jnp.bfloat16),
            pltpu.VMEM((128, N), jnp.bfloat16),
            pltpu.VMEM((256, N), jnp.bfloat16),
            pltpu.VMEM((256, N), jnp.bfloat16),
            pltpu.SemaphoreType.DMA((24,)),
            pltpu.SemaphoreType.DMA((24,)),
        ],
        compiler_params=pltpu.CompilerParams(collective_id=0),
    )(x)
